# baseline (speedup 1.0000x reference)
import math

import jax
import jax.numpy as jnp
from jax.experimental import pallas as pl
from jax.experimental.pallas import tpu as pltpu

N = 4096
D = 1024
B = 512
G = N // B
SIGMA = 0.2
OMEGA = 1.0
EPS = 1e-12
K_F = float(N)
COEFF = K_F / 2.0 - 1.0
KA = 1.0 / (2.0 * K_F * SIGMA * SIGMA)
KB = 1.0 / (2.0 * K_F * OMEGA * OMEGA)
LOG_KA = math.log(KA)
LOG_KB = math.log(KB)
LN2 = math.log(2.0)
C1_SAME = -COEFF * LN2
C1_DIFF = COEFF * LN2
C2_SAME = KA
C2_DIFF = -KB
C0_SAME = -COEFF * (LN2 + LOG_KA)
C0_DIFF = COEFF * (LN2 + LOG_KB)


def _norms_body(x_ref, xb_ref, sq_ref):
    x = x_ref[...]
    xb_ref[...] = x.astype(jnp.bfloat16)
    sq_ref[0, 0, :] = 0.5 * jnp.sum(x * x, axis=1)


def _loss_body(xr_ref, xc_ref, sqr_ref, sqc_ref, lr_ref, lc_ref, out_ref):
    gi = pl.program_id(0)
    gj = pl.program_id(1)

    @pl.when(gj == 0)
    def _init():
        out_ref[...] = jnp.zeros_like(out_ref)

    active = jnp.logical_or(gj < G // 2, gi < G // 2)

    @pl.when(active)
    def _compute():
        gram = jax.lax.dot_general(
            xr_ref[...], xc_ref[...], (((1,), (1,)), ((), ())),
            preferred_element_type=jnp.float32)
        sqr2 = sqr_ref[0, 0, :]
        sqc2 = sqc_ref[0, 0, :]
        e = (sqr2[:, None] + sqc2[None, :]) - gram
        me = jnp.maximum(e, 0.5 * EPS)
        t = jnp.log2(me)
        same = lr_ref[0, 0, :][:, None] == lc_ref[0, 0, :][None, :]
        c1 = jnp.where(same, C1_SAME, C1_DIFF)
        c2 = jnp.where(same, C2_SAME, C2_DIFF)
        c0 = jnp.where(same, C0_SAME, C0_DIFF)
        per = c1 * t + (c2 * me + c0)
        rows = jax.lax.broadcasted_iota(jnp.int32, (B, B), 0)
        cols = jax.lax.broadcasted_iota(jnp.int32, (B, B), 1)
        keep = jnp.logical_or(gj > 0, cols > rows)
        per = jnp.where(keep, per, 0.0)
        colsum = jnp.sum(per, axis=0)
        out_ref[0, 0, :] += jnp.sum(colsum.reshape(B // 128, 128), axis=0)


@jax.jit
def kernel(outputs, labels):
    labels2 = labels.astype(jnp.int32).reshape(G, 1, B)
    xb, sq2 = pl.pallas_call(
        _norms_body,
        grid=(G,),
        in_specs=[pl.BlockSpec((B, D), lambda i: (i, 0))],
        out_specs=[
            pl.BlockSpec((B, D), lambda i: (i, 0)),
            pl.BlockSpec((1, 1, B), lambda i: (i, 0, 0)),
        ],
        out_shape=[
            jax.ShapeDtypeStruct((N, D), jnp.bfloat16),
            jax.ShapeDtypeStruct((G, 1, B), jnp.float32),
        ],
        compiler_params=pltpu.CompilerParams(
            dimension_semantics=("parallel",)),
    )(outputs)
    partials = pl.pallas_call(
        _loss_body,
        grid=(G, G // 2 + 1),
        in_specs=[
            pl.BlockSpec((B, D), lambda i, j: (i, 0)),
            pl.BlockSpec((B, D), lambda i, j: ((i + j) % G, 0)),
            pl.BlockSpec((1, 1, B), lambda i, j: (i, 0, 0)),
            pl.BlockSpec((1, 1, B), lambda i, j: ((i + j) % G, 0, 0)),
            pl.BlockSpec((1, 1, B), lambda i, j: (i, 0, 0)),
            pl.BlockSpec((1, 1, B), lambda i, j: ((i + j) % G, 0, 0)),
        ],
        out_specs=pl.BlockSpec((1, 1, 128), lambda i, j: (i, 0, 0)),
        out_shape=jax.ShapeDtypeStruct((G, 1, 128), jnp.float32),
        compiler_params=pltpu.CompilerParams(
            dimension_semantics=("parallel", "arbitrary")),
    )(xb, xb, sq2, sq2, labels2, labels2)
    return jnp.sum(partials)


# speedup vs baseline: 1.0353x; 1.0353x over previous
import math

import jax
import jax.numpy as jnp
from jax.experimental import pallas as pl
from jax.experimental.pallas import tpu as pltpu

N = 4096
D = 1024
B = 512
G = N // B
SIGMA = 0.2
OMEGA = 1.0
EPS = 1e-12
K_F = float(N)
COEFF = K_F / 2.0 - 1.0
KA = 1.0 / (2.0 * K_F * SIGMA * SIGMA)
KB = 1.0 / (2.0 * K_F * OMEGA * OMEGA)
LOG_KA = math.log(KA)
LOG_KB = math.log(KB)
LN2 = math.log(2.0)
C1_SAME = -COEFF * LN2
C1_DIFF = COEFF * LN2
C2_SAME = KA
C2_DIFF = -KB
C0_SAME = -COEFF * (LN2 + LOG_KA)
C0_DIFF = COEFF * (LN2 + LOG_KB)


def _norms_body(x_ref, xb_ref, sq_ref):
    x = x_ref[...]
    xb_ref[...] = x.astype(jnp.bfloat16)
    sq_ref[0, 0, :] = 0.5 * jnp.sum(x * x, axis=1)


BC = 256


def _loss_body(xr_ref, xc_ref, sqr_ref, sqc_ref, lr_ref, lc_ref, out_ref):
    gi = pl.program_id(0)
    gj = pl.program_id(1)

    @pl.when(gj == 0)
    def _init():
        out_ref[...] = jnp.zeros_like(out_ref)

    def emit(masked):
        xr = xr_ref[...]
        sqr2 = sqr_ref[0, 0, :]
        lr = lr_ref[0, 0, :]
        acc = jnp.zeros((128,), jnp.float32)
        for ci in range(B // BC):
            sl = pl.ds(ci * BC, BC)
            gram = jax.lax.dot_general(
                xr, xc_ref[sl, :], (((1,), (1,)), ((), ())),
                preferred_element_type=jnp.float32)
            sqc2 = sqc_ref[0, 0, sl]
            e = (sqr2[:, None] + sqc2[None, :]) - gram
            me = jnp.maximum(e, 0.5 * EPS)
            t = jnp.log2(me)
            same = lr[:, None] == lc_ref[0, 0, sl][None, :]
            c1 = jnp.where(same, C1_SAME, C1_DIFF)
            c2 = jnp.where(same, C2_SAME, C2_DIFF)
            c0 = jnp.where(same, C0_SAME, C0_DIFF)
            per = c1 * t + (c2 * me + c0)
            if masked:
                rows = jax.lax.broadcasted_iota(jnp.int32, (B, BC), 0)
                cols = jax.lax.broadcasted_iota(jnp.int32, (B, BC), 1)
                per = jnp.where(cols + ci * BC > rows, per, 0.0)
            colsum = jnp.sum(per, axis=0)
            acc = acc + jnp.sum(colsum.reshape(BC // 128, 128), axis=0)
        out_ref[0, 0, :] += acc

    @pl.when(gj == 0)
    def _diag():
        emit(masked=True)

    @pl.when(jnp.logical_and(gj > 0,
                             jnp.logical_or(gj < G // 2, gi < G // 2)))
    def _offdiag():
        emit(masked=False)


@jax.jit
def kernel(outputs, labels):
    labels2 = labels.astype(jnp.int32).reshape(G, 1, B)
    xb, sq2 = pl.pallas_call(
        _norms_body,
        grid=(G,),
        in_specs=[pl.BlockSpec((B, D), lambda i: (i, 0))],
        out_specs=[
            pl.BlockSpec((B, D), lambda i: (i, 0)),
            pl.BlockSpec((1, 1, B), lambda i: (i, 0, 0)),
        ],
        out_shape=[
            jax.ShapeDtypeStruct((N, D), jnp.bfloat16),
            jax.ShapeDtypeStruct((G, 1, B), jnp.float32),
        ],
        compiler_params=pltpu.CompilerParams(
            dimension_semantics=("parallel",)),
    )(outputs)
    partials = pl.pallas_call(
        _loss_body,
        grid=(G, G // 2 + 1),
        in_specs=[
            pl.BlockSpec((B, D), lambda i, j: (i, 0)),
            pl.BlockSpec((B, D), lambda i, j: ((i + j) % G, 0)),
            pl.BlockSpec((1, 1, B), lambda i, j: (i, 0, 0)),
            pl.BlockSpec((1, 1, B), lambda i, j: ((i + j) % G, 0, 0)),
            pl.BlockSpec((1, 1, B), lambda i, j: (i, 0, 0)),
            pl.BlockSpec((1, 1, B), lambda i, j: ((i + j) % G, 0, 0)),
        ],
        out_specs=pl.BlockSpec((1, 1, 128), lambda i, j: (i, 0, 0)),
        out_shape=jax.ShapeDtypeStruct((G, 1, 128), jnp.float32),
        compiler_params=pltpu.CompilerParams(
            dimension_semantics=("parallel", "arbitrary")),
    )(xb, xb, sq2, sq2, labels2, labels2)
    return jnp.sum(partials)
